# baseline (speedup 1.0000x reference)
import functools

import jax
import jax.numpy as jnp
from jax.experimental import pallas as pl


def _adjacency_block(refs_blk_ref, refs_ref, srcs_ref, out_ref, *, block_n):
    i = pl.program_id(1)

    refs_blk = refs_blk_ref[0]
    refs = refs_ref[0]
    srcs = srcs_ref[0]

    sq_all = jnp.sum(refs * refs, axis=0, keepdims=True)
    sq_blk = jnp.sum(refs_blk * refs_blk, axis=0)[:, None]

    inner = jax.lax.dot_general(
        refs_blk, refs,
        dimension_numbers=(((0,), (0,)), ((), ())),
        preferred_element_type=jnp.float32,
    )
    w = jnp.exp(2.0 * inner - sq_blk - sq_all)

    filt = jax.lax.dot_general(
        srcs, w,
        dimension_numbers=(((1,), (1,)), ((), ())),
        preferred_element_type=jnp.float32,
    )
    out_ref[0] = filt - srcs_ref[0, :, pl.ds(i * block_n, block_n)]


def kernel(src_imgs, guide_imgs):
    bs, L, h, w = src_imgs.shape
    d = guide_imgs.shape[1]
    n = h * w

    flat_srcs = src_imgs.reshape(bs, L, n)
    dp = 8
    flat_refs = jnp.zeros((bs, dp, n), jnp.float32).at[:, :d, :].set(
        guide_imgs.reshape(bs, d, n))

    block_n = 512
    grid = (bs, n // block_n)

    out = pl.pallas_call(
        functools.partial(_adjacency_block, block_n=block_n),
        grid=grid,
        in_specs=[
            pl.BlockSpec((1, dp, block_n), lambda b, i: (b, 0, i)),
            pl.BlockSpec((1, dp, n), lambda b, i: (b, 0, 0)),
            pl.BlockSpec((1, L, n), lambda b, i: (b, 0, 0)),
        ],
        out_specs=pl.BlockSpec((1, L, block_n), lambda b, i: (b, 0, i)),
        out_shape=jax.ShapeDtypeStruct((bs, L, n), jnp.float32),
    )(flat_refs, flat_refs, flat_srcs)

    return out.reshape(bs, L, h, w)


# speedup vs baseline: 1.1164x; 1.1164x over previous
import functools

import jax
import jax.numpy as jnp
from jax.experimental import pallas as pl
from jax.experimental.pallas import tpu as pltpu


def _adjacency_block(a_blk_ref, b_ref, srcs_ref, out_ref, *, block_n):
    i = pl.program_id(1)

    neg_d2 = jax.lax.dot_general(
        a_blk_ref[0], b_ref[0],
        dimension_numbers=(((0,), (0,)), ((), ())),
        preferred_element_type=jnp.float32,
    )
    w = jnp.exp2(neg_d2)

    filt = jax.lax.dot_general(
        srcs_ref[0], w,
        dimension_numbers=(((1,), (1,)), ((), ())),
        preferred_element_type=jnp.float32,
    )
    out_ref[0] = filt - srcs_ref[0, :, pl.ds(i * block_n, block_n)]


def kernel(src_imgs, guide_imgs):
    bs, L, h, w = src_imgs.shape
    d = guide_imgs.shape[1]
    n = h * w

    flat_srcs = src_imgs.reshape(bs, L, n)
    refs = guide_imgs.reshape(bs, d, n)

    sq = jnp.sum(refs * refs, axis=1, keepdims=True)
    ones = jnp.ones_like(sq)
    zero = jnp.zeros_like(sq)
    log2e = jnp.float32(1.4426950408889634)
    a_aug = log2e * jnp.concatenate([2.0 * refs, -sq, ones, zero], axis=1)
    b_aug = jnp.concatenate([refs, ones, -sq, zero], axis=1)
    dp = d + 3

    block_n = 512
    grid = (bs, n // block_n)

    out = pl.pallas_call(
        functools.partial(_adjacency_block, block_n=block_n),
        grid=grid,
        in_specs=[
            pl.BlockSpec((1, dp, block_n), lambda b, i: (b, 0, i)),
            pl.BlockSpec((1, dp, n), lambda b, i: (b, 0, 0)),
            pl.BlockSpec((1, L, n), lambda b, i: (b, 0, 0)),
        ],
        out_specs=pl.BlockSpec((1, L, block_n), lambda b, i: (b, 0, i)),
        out_shape=jax.ShapeDtypeStruct((bs, L, n), jnp.float32),
        compiler_params=pltpu.CompilerParams(
            dimension_semantics=("parallel", "parallel")),
    )(a_aug, b_aug, flat_srcs)

    return out.reshape(bs, L, h, w)
